# Initial kernel scaffold; baseline (speedup 1.0000x reference)
import functools

import jax
import jax.numpy as jnp
from jax import lax
from jax.experimental import pallas as pl
from jax.experimental.pallas import tpu as pltpu
from jax.experimental.pallas import tpu_sc as plsc

N = 10000
E = 320000
F = 128
H = 128
C = 2

NC = 2
NS = 16
NW = NC * NS
EP = E // NW
K = 80
NCHUNK = EP // K
RPT = N // NS

_MESH = plsc.VectorSubcoreMesh(
    core_axis_name="c", subcore_axis_name="s", num_cores=NC, num_subcores=NS
)


@functools.partial(
    pl.kernel,
    out_type=jax.ShapeDtypeStruct((NC, N, 16), jnp.float32),
    mesh=_MESH,
    scratch_types=[
        pltpu.VMEM((K,), jnp.int32),
        pltpu.VMEM((K, 16), jnp.float32),
        pltpu.VMEM((RPT, 16), jnp.float32),
        pltpu.VMEM_SHARED((N, 16), jnp.float32),
        pltpu.SemaphoreType.DMA,
    ],
)
def _deg_kernel(dst_hbm, out_hbm, idx_v, ones_v, zero_v, deg_sh, sem):
    cid = lax.axis_index("c")
    sid = lax.axis_index("s")
    wid = cid * NS + sid

    @pl.loop(0, K)
    def _(i):
        ones_v[i, :] = jnp.full((16,), 1.0, jnp.float32)

    @pl.loop(0, RPT)
    def _(i):
        zero_v[i, :] = jnp.zeros((16,), jnp.float32)

    pltpu.sync_copy(zero_v, deg_sh.at[pl.ds(sid * RPT, RPT)])
    plsc.subcore_barrier()

    @pl.loop(0, NCHUNK)
    def _(c):
        base = wid * EP + c * K
        pltpu.sync_copy(dst_hbm.at[pl.ds(base, K)], idx_v)
        pltpu.sync_copy(ones_v, deg_sh.at[idx_v], add=True)

    plsc.subcore_barrier()
    pltpu.sync_copy(
        deg_sh.at[pl.ds(sid * RPT, RPT)], out_hbm.at[cid, pl.ds(sid * RPT, RPT)]
    )


ZR = 125


@functools.partial(
    pl.kernel,
    out_type=jax.ShapeDtypeStruct((NC, N, F), jnp.float32),
    mesh=_MESH,
    scratch_types=[
        pltpu.VMEM((K,), jnp.int32),
        pltpu.VMEM((K,), jnp.int32),
        pltpu.VMEM((K, F), jnp.float32),
        pltpu.VMEM((ZR, F), jnp.float32),
        pltpu.VMEM_SHARED((N, F), jnp.float32),
        pltpu.SemaphoreType.DMA,
    ],
)
def _scatter_kernel(y_hbm, src_hbm, dst_hbm, out_hbm, sidx, didx, rows, zbuf, acc_sh, sem):
    cid = lax.axis_index("c")
    sid = lax.axis_index("s")
    wid = cid * NS + sid

    @pl.loop(0, ZR)
    def _(i):
        @pl.loop(0, F // 16)
        def _(j):
            zbuf[i, pl.ds(j * 16, 16)] = jnp.zeros((16,), jnp.float32)

    @pl.loop(0, RPT // ZR)
    def _(b):
        pltpu.sync_copy(zbuf, acc_sh.at[pl.ds(sid * RPT + b * ZR, ZR)])

    plsc.subcore_barrier()

    @pl.loop(0, NCHUNK)
    def _(c):
        base = wid * EP + c * K
        pltpu.sync_copy(src_hbm.at[pl.ds(base, K)], sidx)
        pltpu.sync_copy(dst_hbm.at[pl.ds(base, K)], didx)
        pltpu.async_copy(y_hbm.at[sidx], rows, sem).wait()
        pltpu.sync_copy(rows, acc_sh.at[didx], add=True)

    plsc.subcore_barrier()
    pltpu.sync_copy(
        acc_sh.at[pl.ds(sid * RPT, RPT)], out_hbm.at[cid, pl.ds(sid * RPT, RPT)]
    )


RB = 1000


def _mm1_body(x_ref, w_ref, o_ref):
    o_ref[...] = jnp.dot(x_ref[...], w_ref[...], preferred_element_type=jnp.float32)


def _scale_body(xw_ref, deg_ref, y_ref):
    d = deg_ref[0, :, 0] + deg_ref[1, :, 0] + 1.0
    dinv = lax.rsqrt(d)
    y_ref[...] = xw_ref[...] * dinv[:, None]


def _final_body(x_ref, acc_ref, y_ref, deg_ref, wa_ref, wb_ref, b1_ref, bfc_ref, o_ref):
    d = deg_ref[0, :, 0] + deg_ref[1, :, 0] + 1.0
    dinv = lax.rsqrt(d)
    agg = (acc_ref[0] + acc_ref[1] + y_ref[...]) * dinv[:, None]
    h = jnp.maximum(agg + b1_ref[...], 0.0)
    o_ref[...] = (
        jnp.dot(x_ref[...], wa_ref[...], preferred_element_type=jnp.float32)
        + jnp.dot(h, wb_ref[...], preferred_element_type=jnp.float32)
        + bfc_ref[...]
    )


def kernel(x, edge_index, W1, b1, Wfc, bfc):
    src = edge_index[0]
    dst = edge_index[1]

    degp = _deg_kernel(dst)

    xw = pl.pallas_call(
        _mm1_body,
        grid=(N // RB,),
        in_specs=[
            pl.BlockSpec((RB, F), lambda i: (i, 0)),
            pl.BlockSpec((F, H), lambda i: (0, 0)),
        ],
        out_specs=pl.BlockSpec((RB, H), lambda i: (i, 0)),
        out_shape=jax.ShapeDtypeStruct((N, H), jnp.float32),
    )(x, W1)

    y = pl.pallas_call(
        _scale_body,
        grid=(N // RB,),
        in_specs=[
            pl.BlockSpec((RB, H), lambda i: (i, 0)),
            pl.BlockSpec((NC, RB, 16), lambda i: (0, i, 0)),
        ],
        out_specs=pl.BlockSpec((RB, H), lambda i: (i, 0)),
        out_shape=jax.ShapeDtypeStruct((N, H), jnp.float32),
    )(xw, degp)

    accp = _scatter_kernel(y, src, dst)

    wa = Wfc[:F]
    wb = Wfc[F:]
    b1r = b1.reshape(1, H)
    bfcr = bfc.reshape(1, C)

    out = pl.pallas_call(
        _final_body,
        grid=(N // RB,),
        in_specs=[
            pl.BlockSpec((RB, F), lambda i: (i, 0)),
            pl.BlockSpec((NC, RB, H), lambda i: (0, i, 0)),
            pl.BlockSpec((RB, H), lambda i: (i, 0)),
            pl.BlockSpec((NC, RB, 16), lambda i: (0, i, 0)),
            pl.BlockSpec((F, C), lambda i: (0, 0)),
            pl.BlockSpec((H, C), lambda i: (0, 0)),
            pl.BlockSpec((1, H), lambda i: (0, 0)),
            pl.BlockSpec((1, C), lambda i: (0, 0)),
        ],
        out_specs=pl.BlockSpec((RB, C), lambda i: (i, 0)),
        out_shape=jax.ShapeDtypeStruct((N, C), jnp.float32),
    )(x, accp, y, degp, wa, wb, b1r, bfcr)

    return out


# speedup vs baseline: 1.7918x; 1.7918x over previous
import dataclasses
import functools

import jax
import jax.numpy as jnp
from jax import lax
from jax.experimental import pallas as pl
from jax.experimental.pallas import tpu as pltpu
from jax.experimental.pallas import tpu_sc as plsc

N = 10000
E = 320000
F = 128
H = 128
C = 2

NC = 2
NS = 16
NW = NC * NS
EP = E // NW
K = 80
NCHUNK = EP // K
NP = 10240
RPT = NP // NS
ESC = E // NC
NGRP = K // 16
NSCAN = ESC // K
LCAP = 1024
FB = 128

_MESH = plsc.VectorSubcoreMesh(
    core_axis_name="c", subcore_axis_name="s", num_cores=NC, num_subcores=NS
)

_SC_PARAMS = pltpu.CompilerParams()
if "needs_layout_passes" in pltpu.CompilerParams.__dataclass_fields__:
    _SC_PARAMS = dataclasses.replace(_SC_PARAMS, needs_layout_passes=False)


@functools.partial(
    pl.kernel,
    out_type=jax.ShapeDtypeStruct((NW, NP), jnp.float32),
    mesh=_MESH,
    scratch_types=[
        pltpu.VMEM((K,), jnp.int32),
        pltpu.VMEM((NP,), jnp.float32),
        pltpu.SemaphoreType.DMA,
    ],
    compiler_params=_SC_PARAMS,
)
def _deg_kernel(dst_hbm, out_hbm, idx_v, deg_v, sem):
    cid = lax.axis_index("c")
    sid = lax.axis_index("s")
    wid = cid * NS + sid

    @pl.loop(0, NP // 16)
    def _(i):
        deg_v[pl.ds(i * 16, 16)] = jnp.zeros((16,), jnp.float32)

    ones16 = jnp.full((16,), 1.0, jnp.float32)

    @pl.loop(0, NCHUNK)
    def _(c):
        pltpu.sync_copy(dst_hbm.at[pl.ds(wid * EP + c * K, K)], idx_v)

        @pl.loop(0, NGRP)
        def _(g):
            idx16 = idx_v[pl.ds(g * 16, 16)]
            plsc.addupdate_scatter(deg_v, [idx16], ones16)

    pltpu.sync_copy(deg_v, out_hbm.at[wid])


@functools.partial(
    pl.kernel,
    out_type=jax.ShapeDtypeStruct((NC, NP, F), jnp.float32),
    mesh=_MESH,
    scratch_types=[
        pltpu.VMEM((K,), jnp.int32),
        pltpu.VMEM((K,), jnp.int32),
        pltpu.VMEM((LCAP,), jnp.int32),
        pltpu.VMEM((LCAP,), jnp.int32),
        pltpu.VMEM((FB, F), jnp.float32),
        pltpu.VMEM((RPT, F), jnp.float32),
        pltpu.SemaphoreType.DMA,
    ],
    compiler_params=_SC_PARAMS,
)
def _scatter_kernel(y_hbm, src_hbm, dst_hbm, out_hbm, sidx, didx, slist, dlist,
                    rows_v, acc_v, sem):
    cid = lax.axis_index("c")
    sid = lax.axis_index("s")
    lo = sid * RPT

    @pl.loop(0, RPT)
    def _(i):
        @pl.loop(0, F // 16)
        def _(j):
            acc_v[i, pl.ds(j * 16, 16)] = jnp.zeros((16,), jnp.float32)

    @pl.loop(0, LCAP // 16)
    def _(i):
        slist[pl.ds(i * 16, 16)] = jnp.zeros((16,), jnp.int32)
        dlist[pl.ds(i * 16, 16)] = jnp.zeros((16,), jnp.int32)

    iota16 = lax.iota(jnp.int32, 16)

    def flush(cur):
        nbat = (cur + FB - 1) // FB

        def batch(b, _):
            base = b * FB
            nrows = cur - base
            pltpu.async_copy(
                y_hbm.at[slist.at[pl.ds(base, FB)]], rows_v, sem
            ).wait()

            @pl.loop(0, FB // 16)
            def _(g):
                rem = nrows - g * 16
                m = iota16 < rem
                dl16 = dlist[pl.ds(base + g * 16, 16)]
                row16 = iota16 + g * 16

                @pl.loop(0, F)
                def _(c):
                    cc = jnp.full((16,), c, jnp.int32)
                    val = plsc.load_gather(rows_v, [row16, cc])
                    plsc.addupdate_scatter(acc_v, [dl16, cc], val, mask=m)

            return 0

        lax.fori_loop(0, nbat, batch, 0)

    def scan_body(t, cur):
        base = cid * ESC + t * K
        pltpu.sync_copy(src_hbm.at[pl.ds(base, K)], sidx)
        pltpu.sync_copy(dst_hbm.at[pl.ds(base, K)], didx)

        def group(g, cur):
            s16 = sidx[pl.ds(g * 16, 16)]
            d16 = didx[pl.ds(g * 16, 16)]
            dloc = d16 - lo
            m = (dloc >= 0) & (dloc < RPT)
            plsc.store_compressed(slist.at[pl.ds(cur, 16)], s16, mask=m)
            plsc.store_compressed(dlist.at[pl.ds(cur, 16)], dloc, mask=m)
            cnt = plsc.all_reduce_population_count(m)[0]
            return cur + cnt

        cur = lax.fori_loop(0, NGRP, group, cur)
        full_now = cur > LCAP - K

        @pl.when(full_now)
        def _():
            flush(cur)

        return jnp.where(full_now, 0, cur)

    cur = lax.fori_loop(0, NSCAN, scan_body, 0)
    flush(cur)

    @pl.loop(0, RPT // FB)
    def _(b):
        pltpu.sync_copy(
            acc_v.at[pl.ds(b * FB, FB)],
            out_hbm.at[cid, pl.ds(lo + b * FB, FB)],
        )


RB = 1280


def _mm1_body(x_ref, w_ref, o_ref):
    o_ref[...] = jnp.dot(x_ref[...], w_ref[...], preferred_element_type=jnp.float32)


def _scale_body(xw_ref, deg_ref, y_ref):
    d = jnp.sum(deg_ref[...], axis=0) + 1.0
    dinv = lax.rsqrt(d)
    y_ref[...] = xw_ref[...] * dinv[:, None]


def _final_body(x_ref, acc_ref, y_ref, deg_ref, wa_ref, wb_ref, b1_ref, bfc_ref, o_ref):
    d = jnp.sum(deg_ref[...], axis=0) + 1.0
    dinv = lax.rsqrt(d)
    agg = (acc_ref[0] + acc_ref[1] + y_ref[...]) * dinv[:, None]
    h = jnp.maximum(agg + b1_ref[...], 0.0)
    o_ref[...] = (
        jnp.dot(x_ref[...], wa_ref[...], preferred_element_type=jnp.float32)
        + jnp.dot(h, wb_ref[...], preferred_element_type=jnp.float32)
        + bfc_ref[...]
    )


def kernel(x, edge_index, W1, b1, Wfc, bfc):
    src = edge_index[0]
    dst = edge_index[1]

    degp = _deg_kernel(dst)

    xw = pl.pallas_call(
        _mm1_body,
        grid=(NP // RB,),
        in_specs=[
            pl.BlockSpec((RB, F), lambda i: (i, 0)),
            pl.BlockSpec((F, H), lambda i: (0, 0)),
        ],
        out_specs=pl.BlockSpec((RB, H), lambda i: (i, 0)),
        out_shape=jax.ShapeDtypeStruct((N, H), jnp.float32),
    )(x, W1)

    y = pl.pallas_call(
        _scale_body,
        grid=(NP // RB,),
        in_specs=[
            pl.BlockSpec((RB, H), lambda i: (i, 0)),
            pl.BlockSpec((NW, RB), lambda i: (0, i)),
        ],
        out_specs=pl.BlockSpec((RB, H), lambda i: (i, 0)),
        out_shape=jax.ShapeDtypeStruct((N, H), jnp.float32),
    )(xw, degp)

    accp = _scatter_kernel(y, src, dst)

    wa = Wfc[:F]
    wb = Wfc[F:]
    b1r = b1.reshape(1, H)
    bfcr = bfc.reshape(1, C)

    out = pl.pallas_call(
        _final_body,
        grid=(NP // RB,),
        in_specs=[
            pl.BlockSpec((RB, F), lambda i: (i, 0)),
            pl.BlockSpec((NC, RB, H), lambda i: (0, i, 0)),
            pl.BlockSpec((RB, H), lambda i: (i, 0)),
            pl.BlockSpec((NW, RB), lambda i: (0, i)),
            pl.BlockSpec((F, C), lambda i: (0, 0)),
            pl.BlockSpec((H, C), lambda i: (0, 0)),
            pl.BlockSpec((1, H), lambda i: (0, 0)),
            pl.BlockSpec((1, C), lambda i: (0, 0)),
        ],
        out_specs=pl.BlockSpec((RB, C), lambda i: (i, 0)),
        out_shape=jax.ShapeDtypeStruct((N, C), jnp.float32),
    )(x, accp, y, degp, wa, wb, b1r, bfcr)

    return out


# speedup vs baseline: 2.9084x; 1.6232x over previous
import dataclasses
import functools

import jax
import jax.numpy as jnp
from jax import lax
from jax.experimental import pallas as pl
from jax.experimental.pallas import tpu as pltpu
from jax.experimental.pallas import tpu_sc as plsc

N = 10000
E = 320000
F = 128
H = 128
C = 2

NC = 2
NS = 16
NW = NC * NS
EP = E // NW
K = 2000
NCHUNK = EP // K
KS = 1280
NP = 10240
RPT = NP // NS
ESC = E // NC
NGRP = K // 16
NGRPS = KS // 16
NSCAN = ESC // KS
LCAP = 2048
FB = 128

_MESH = plsc.VectorSubcoreMesh(
    core_axis_name="c", subcore_axis_name="s", num_cores=NC, num_subcores=NS
)

_SC_PARAMS = pltpu.CompilerParams()
if "needs_layout_passes" in pltpu.CompilerParams.__dataclass_fields__:
    _SC_PARAMS = dataclasses.replace(_SC_PARAMS, needs_layout_passes=False)


@functools.partial(
    pl.kernel,
    out_type=jax.ShapeDtypeStruct((NW, NP), jnp.float32),
    mesh=_MESH,
    scratch_types=[
        pltpu.VMEM((K,), jnp.int32),
        pltpu.VMEM((NP,), jnp.float32),
        pltpu.SemaphoreType.DMA,
    ],
    compiler_params=_SC_PARAMS,
)
def _deg_kernel(dst_hbm, out_hbm, idx_v, deg_v, sem):
    cid = lax.axis_index("c")
    sid = lax.axis_index("s")
    wid = cid * NS + sid

    @pl.loop(0, NP // 16)
    def _(i):
        deg_v[pl.ds(i * 16, 16)] = jnp.zeros((16,), jnp.float32)

    ones16 = jnp.full((16,), 1.0, jnp.float32)

    @pl.loop(0, NCHUNK)
    def _(c):
        pltpu.sync_copy(dst_hbm.at[pl.ds(wid * EP + c * K, K)], idx_v)

        @pl.loop(0, NGRP)
        def _(g):
            idx16 = idx_v[pl.ds(g * 16, 16)]
            plsc.addupdate_scatter(deg_v, [idx16], ones16)

    pltpu.sync_copy(deg_v, out_hbm.at[wid])


@functools.partial(
    pl.kernel,
    out_type=jax.ShapeDtypeStruct((NC, NP, F), jnp.float32),
    mesh=_MESH,
    scratch_types=[
        pltpu.VMEM((KS,), jnp.int32),
        pltpu.VMEM((KS,), jnp.int32),
        pltpu.VMEM((LCAP,), jnp.int32),
        pltpu.VMEM((LCAP,), jnp.int32),
        pltpu.VMEM((FB, F), jnp.float32),
        pltpu.VMEM((RPT, F), jnp.float32),
        pltpu.SemaphoreType.DMA,
    ],
    compiler_params=_SC_PARAMS,
)
def _scatter_kernel(y_hbm, src_hbm, dst_hbm, out_hbm, sidx, didx, slist, dlist,
                    rows_v, acc_v, sem):
    cid = lax.axis_index("c")
    sid = lax.axis_index("s")
    lo = sid * RPT

    @pl.loop(0, RPT)
    def _(i):
        @pl.loop(0, F // 16)
        def _(j):
            acc_v[i, pl.ds(j * 16, 16)] = jnp.zeros((16,), jnp.float32)

    @pl.loop(0, LCAP // 16)
    def _(i):
        slist[pl.ds(i * 16, 16)] = jnp.zeros((16,), jnp.int32)
        dlist[pl.ds(i * 16, 16)] = jnp.zeros((16,), jnp.int32)

    iota16 = lax.iota(jnp.int32, 16)

    def flush(cur):
        nbat = (cur + FB - 1) // FB

        def batch(b, _):
            base = b * FB
            nrows = cur - base
            pltpu.async_copy(
                y_hbm.at[slist.at[pl.ds(base, FB)]], rows_v, sem
            ).wait()

            @pl.loop(0, FB // 16)
            def _(g):
                rem = nrows - g * 16
                m = iota16 < rem
                dl16 = dlist[pl.ds(base + g * 16, 16)]
                row16 = iota16 + g * 16

                @pl.loop(0, F)
                def _(c):
                    cc = jnp.full((16,), c, jnp.int32)
                    val = plsc.load_gather(rows_v, [row16, cc])
                    plsc.addupdate_scatter(acc_v, [dl16, cc], val, mask=m)

            return 0

        lax.fori_loop(0, nbat, batch, 0)

    def scan_body(t, cur):
        base = cid * ESC + t * KS
        pltpu.sync_copy(src_hbm.at[pl.ds(base, KS)], sidx)
        pltpu.sync_copy(dst_hbm.at[pl.ds(base, KS)], didx)

        def group(g, cur):
            s16 = sidx[pl.ds(g * 16, 16)]
            d16 = didx[pl.ds(g * 16, 16)]
            dloc = d16 - lo
            m = (dloc >= 0) & (dloc < RPT)
            plsc.store_compressed(slist.at[pl.ds(cur, 16)], s16, mask=m)
            plsc.store_compressed(dlist.at[pl.ds(cur, 16)], dloc, mask=m)
            cnt = plsc.all_reduce_population_count(m)[0]
            return cur + cnt

        cur = lax.fori_loop(0, NGRPS, group, cur)
        full_now = cur > LCAP - KS

        @pl.when(full_now)
        def _():
            flush(cur)

        return jnp.where(full_now, 0, cur)

    cur = lax.fori_loop(0, NSCAN, scan_body, 0)
    flush(cur)

    @pl.loop(0, RPT // FB)
    def _(b):
        pltpu.sync_copy(
            acc_v.at[pl.ds(b * FB, FB)],
            out_hbm.at[cid, pl.ds(lo + b * FB, FB)],
        )


RB = 1280


def _mm1_body(x_ref, w_ref, o_ref):
    o_ref[...] = jnp.dot(x_ref[...], w_ref[...], preferred_element_type=jnp.float32)


def _scale_body(xw_ref, deg_ref, y_ref):
    d = jnp.sum(deg_ref[...], axis=0) + 1.0
    dinv = lax.rsqrt(d)
    y_ref[...] = xw_ref[...] * dinv[:, None]


def _final_body(x_ref, acc_ref, y_ref, deg_ref, wa_ref, wb_ref, b1_ref, bfc_ref, o_ref):
    d = jnp.sum(deg_ref[...], axis=0) + 1.0
    dinv = lax.rsqrt(d)
    agg = (acc_ref[0] + acc_ref[1] + y_ref[...]) * dinv[:, None]
    h = jnp.maximum(agg + b1_ref[...], 0.0)
    o_ref[...] = (
        jnp.dot(x_ref[...], wa_ref[...], preferred_element_type=jnp.float32)
        + jnp.dot(h, wb_ref[...], preferred_element_type=jnp.float32)
        + bfc_ref[...]
    )


def kernel(x, edge_index, W1, b1, Wfc, bfc):
    src = edge_index[0]
    dst = edge_index[1]

    degp = _deg_kernel(dst)

    xw = pl.pallas_call(
        _mm1_body,
        grid=(NP // RB,),
        in_specs=[
            pl.BlockSpec((RB, F), lambda i: (i, 0)),
            pl.BlockSpec((F, H), lambda i: (0, 0)),
        ],
        out_specs=pl.BlockSpec((RB, H), lambda i: (i, 0)),
        out_shape=jax.ShapeDtypeStruct((N, H), jnp.float32),
    )(x, W1)

    y = pl.pallas_call(
        _scale_body,
        grid=(NP // RB,),
        in_specs=[
            pl.BlockSpec((RB, H), lambda i: (i, 0)),
            pl.BlockSpec((NW, RB), lambda i: (0, i)),
        ],
        out_specs=pl.BlockSpec((RB, H), lambda i: (i, 0)),
        out_shape=jax.ShapeDtypeStruct((N, H), jnp.float32),
    )(xw, degp)

    accp = _scatter_kernel(y, src, dst)

    wa = Wfc[:F]
    wb = Wfc[F:]
    b1r = b1.reshape(1, H)
    bfcr = bfc.reshape(1, C)

    out = pl.pallas_call(
        _final_body,
        grid=(NP // RB,),
        in_specs=[
            pl.BlockSpec((RB, F), lambda i: (i, 0)),
            pl.BlockSpec((NC, RB, H), lambda i: (0, i, 0)),
            pl.BlockSpec((RB, H), lambda i: (i, 0)),
            pl.BlockSpec((NW, RB), lambda i: (0, i)),
            pl.BlockSpec((F, C), lambda i: (0, 0)),
            pl.BlockSpec((H, C), lambda i: (0, 0)),
            pl.BlockSpec((1, H), lambda i: (0, 0)),
            pl.BlockSpec((1, C), lambda i: (0, 0)),
        ],
        out_specs=pl.BlockSpec((RB, C), lambda i: (i, 0)),
        out_shape=jax.ShapeDtypeStruct((N, C), jnp.float32),
    )(x, accp, y, degp, wa, wb, b1r, bfcr)

    return out
